# baseline (speedup 1.0000x reference)
import functools

import jax
import jax.numpy as jnp
from jax import lax
from jax.experimental import pallas as pl
from jax.experimental.pallas import tpu as pltpu
from jax.experimental.pallas import tpu_sc as plsc

NC = 2
NS = 16
NW = NC * NS

BATCH = 4096
SEQ = 200
DIM = 64
PAIRS = SEQ // 2
BPW = BATCH // NW
LANES = 16
GROUPS = DIM // LANES
BB = 512


def _sc_body(xT_hbm, pos_hbm, wword_hbm, out_hbm,
             idx_v, pos_v, gb0, gb1, tb0, tb1,
             gsem0, gsem1, wsem0, wsem1):
    c = lax.axis_index("c")
    s = lax.axis_index("s")
    wid = s * NC + c
    b0 = wid * BPW

    gb = (gb0, gb1)
    tb = (tb0, tb1)
    gsem = (gsem0, gsem1)
    wsem = (wsem0, wsem1)

    pltpu.sync_copy(xT_hbm.at[:, pl.ds(b0, BPW)], idx_v)
    pltpu.sync_copy(pos_hbm, pos_v)

    def gather_desc(l, par):
        return pltpu.make_async_copy(wword_hbm.at[idx_v.at[l]], gb[par],
                                     gsem[par])

    def write_desc(pair, tpar):
        return pltpu.make_async_copy(tb[tpar],
                                     out_hbm.at[pair, pl.ds(b0, BPW), :],
                                     wsem[tpar])

    def compute(l, gpar, tpar, half):
        p = [pos_v[pl.ds(DIM * l + LANES * g, LANES)] for g in range(GROUPS)]
        gbuf = gb[gpar]
        tbuf = tb[tpar]
        co = DIM * half

        def add_row(b, carry):
            for g in range(GROUPS):
                tbuf[b, pl.ds(co + LANES * g, LANES)] = (
                    gbuf[b, pl.ds(LANES * g, LANES)] + p[g])
            return carry

        lax.fori_loop(0, BPW, add_row, 0)

    gather_desc(0, 0).start()
    gather_desc(1, 1).start()

    def quad(i, carry):
        for k in range(4):
            l = 4 * i + k
            gpar = k % 2
            tpar = (k // 2) % 2
            gather_desc(l, gpar).wait()
            if k % 2 == 0:
                @pl.when(i >= 1)
                def _():
                    write_desc(l // 2 - 2, tpar).wait()
            compute(l, gpar, tpar, k % 2)
            if k < 2:
                gather_desc(l + 2, gpar).start()
            else:
                @pl.when(l + 2 < SEQ)
                def _():
                    gather_desc(l + 2, gpar).start()
            if k % 2 == 1:
                write_desc(l // 2, tpar).start()
        return carry

    lax.fori_loop(0, SEQ // 4, quad, 0)
    write_desc(PAIRS - 2, 0).wait()
    write_desc(PAIRS - 1, 1).wait()


def _tc_transpose_body(in_ref, out_ref):
    out_ref[...] = in_ref[0].T


@functools.partial(jax.jit, donate_argnums=())
def kernel(x, W_pos, W_word):
    xT = x.T
    pos = W_pos[:SEQ].reshape(-1)
    mesh = plsc.VectorSubcoreMesh(core_axis_name="c", subcore_axis_name="s",
                                  num_cores=NC, num_subcores=NS)
    run = pl.kernel(
        _sc_body,
        out_type=jax.ShapeDtypeStruct((PAIRS, BATCH, 2 * DIM), jnp.float32),
        mesh=mesh,
        scratch_types=[
            pltpu.VMEM((SEQ, BPW), jnp.int32),
            pltpu.VMEM((SEQ * DIM,), jnp.float32),
            pltpu.VMEM((BPW, DIM), jnp.float32),
            pltpu.VMEM((BPW, DIM), jnp.float32),
            pltpu.VMEM((BPW, 2 * DIM), jnp.float32),
            pltpu.VMEM((BPW, 2 * DIM), jnp.float32),
            pltpu.SemaphoreType.DMA,
            pltpu.SemaphoreType.DMA,
            pltpu.SemaphoreType.DMA,
            pltpu.SemaphoreType.DMA,
        ],
        compiler_params=pltpu.CompilerParams(use_tc_tiling_on_sc=False),
    )
    mid = run(xT, pos, W_word)

    outT = pl.pallas_call(
        _tc_transpose_body,
        grid=(PAIRS, BATCH // BB),
        in_specs=[pl.BlockSpec((1, BB, 2 * DIM), lambda P, q: (P, q, 0))],
        out_specs=pl.BlockSpec((2 * DIM, BB), lambda P, q: (P, q)),
        out_shape=jax.ShapeDtypeStruct((PAIRS * 2 * DIM, BATCH), jnp.float32),
    )(mid)

    return jnp.transpose(outT.reshape(SEQ, DIM, BATCH), (2, 0, 1))


# speedup vs baseline: 1.2285x; 1.2285x over previous
import functools

import jax
import jax.numpy as jnp
from jax import lax
from jax.experimental import pallas as pl
from jax.experimental.pallas import tpu as pltpu
from jax.experimental.pallas import tpu_sc as plsc

NC = 2
NS = 16
NW = NC * NS

BATCH = 4096
SEQ = 200
DIM = 64
PAIRS = SEQ // 2
BPW = BATCH // NW
LANES = 16
GROUPS = DIM // LANES
RUNROLL = 4
BB = 512


def _sc_body(xT_hbm, pos_hbm, wword_hbm, out_hbm,
             idx_v, pos_v, gb0, gb1, ob0, ob1,
             gsem0, gsem1, wsem0, wsem1):
    c = lax.axis_index("c")
    s = lax.axis_index("s")
    wid = s * NC + c
    b0 = wid * BPW

    gb = (gb0, gb1)
    ob = (ob0, ob1)
    gsem = (gsem0, gsem1)
    wsem = (wsem0, wsem1)

    pltpu.sync_copy(xT_hbm.at[:, pl.ds(b0, BPW)], idx_v)
    pltpu.sync_copy(pos_hbm, pos_v)

    def gather_desc(l, par):
        return pltpu.make_async_copy(wword_hbm.at[idx_v.at[l]], gb[par],
                                     gsem[par])

    def write_desc(l, par):
        return pltpu.make_async_copy(
            ob[par],
            out_hbm.at[l // 2, pl.ds(b0, BPW), pl.ds(DIM * (l % 2), DIM)],
            wsem[par])

    gather_desc(0, 0).start()
    gather_desc(1, 1).start()

    def step(i, carry):
        for par in range(2):
            l = 2 * i + par
            gather_desc(l, par).wait()
            @pl.when(i >= 1)
            def _():
                write_desc(l - 2, par).wait()
            p = [pos_v[l, pl.ds(LANES * g, LANES)] for g in range(GROUPS)]
            gbuf = gb[par]
            obuf = ob[par]

            def add_block(r4, carry2):
                for rr in range(RUNROLL):
                    r = r4 * RUNROLL + rr
                    for g in range(GROUPS):
                        sl = pl.ds(LANES * g, LANES)
                        obuf[r, sl] = gbuf[r, sl] + p[g]
                return carry2

            lax.fori_loop(0, BPW // RUNROLL, add_block, 0)
            @pl.when(l + 2 < SEQ)
            def _():
                gather_desc(l + 2, par).start()
            write_desc(l, par).start()
        return carry

    lax.fori_loop(0, SEQ // 2, step, 0)
    write_desc(SEQ - 2, 0).wait()
    write_desc(SEQ - 1, 1).wait()


def _tc_transpose_body(in_ref, out_ref):
    blk = in_ref[0]
    r = lax.broadcasted_iota(jnp.int32, (2 * DIM, 2 * DIM), 0)
    cc = lax.broadcasted_iota(jnp.int32, (2 * DIM, 2 * DIM), 1)
    eye = (r == cc).astype(jnp.float32)
    out_ref[...] = lax.dot_general(eye, blk, (((1,), (1,)), ((), ())),
                                   preferred_element_type=jnp.float32)


@functools.partial(jax.jit, donate_argnums=())
def kernel(x, W_pos, W_word):
    xT = x.T
    pos = W_pos[:SEQ]
    mesh = plsc.VectorSubcoreMesh(core_axis_name="c", subcore_axis_name="s",
                                  num_cores=NC, num_subcores=NS)
    run = pl.kernel(
        _sc_body,
        out_type=jax.ShapeDtypeStruct((PAIRS, BATCH, 2 * DIM), jnp.float32),
        mesh=mesh,
        scratch_types=[
            pltpu.VMEM((SEQ, BPW), jnp.int32),
            pltpu.VMEM((SEQ, DIM), jnp.float32),
            pltpu.VMEM((BPW, DIM), jnp.float32),
            pltpu.VMEM((BPW, DIM), jnp.float32),
            pltpu.VMEM((BPW, DIM), jnp.float32),
            pltpu.VMEM((BPW, DIM), jnp.float32),
            pltpu.SemaphoreType.DMA,
            pltpu.SemaphoreType.DMA,
            pltpu.SemaphoreType.DMA,
            pltpu.SemaphoreType.DMA,
        ],
        compiler_params=pltpu.CompilerParams(use_tc_tiling_on_sc=False),
    )
    mid = run(xT, pos, W_word)

    outT = pl.pallas_call(
        _tc_transpose_body,
        grid=(PAIRS, BATCH // BB),
        in_specs=[pl.BlockSpec((1, BB, 2 * DIM), lambda P, q: (P, q, 0))],
        out_specs=pl.BlockSpec((2 * DIM, BB), lambda P, q: (P, q)),
        out_shape=jax.ShapeDtypeStruct((PAIRS * 2 * DIM, BATCH), jnp.float32),
    )(mid)

    return jnp.transpose(outT.reshape(SEQ, DIM, BATCH), (2, 0, 1))


# speedup vs baseline: 1.7228x; 1.4023x over previous
import functools

import jax
import jax.numpy as jnp
from jax import lax
from jax.experimental import pallas as pl
from jax.experimental.pallas import tpu as pltpu
from jax.experimental.pallas import tpu_sc as plsc

NC = 2
NS = 16
NW = NC * NS

BATCH = 4096
SEQ = 200
DIM = 64
PAIRS = SEQ // 2
BPW = BATCH // NW
LANES = 16
GROUPS = DIM // LANES
RUNROLL = 4
BB = 512


def _sc_body(xT_hbm, pos_hbm, wword_hbm, out_hbm,
             idx_v, pos_v, gb0, gb1, ob0, ob1,
             gsem0, gsem1, wsem0, wsem1):
    c = lax.axis_index("c")
    s = lax.axis_index("s")
    wid = s * NC + c
    b0 = wid * BPW

    gb = (gb0, gb1)
    ob = (ob0, ob1)
    gsem = (gsem0, gsem1)
    wsem = (wsem0, wsem1)

    pltpu.sync_copy(xT_hbm.at[:, pl.ds(b0, BPW)], idx_v)
    pltpu.sync_copy(pos_hbm, pos_v)

    def gather_desc(l, par):
        return pltpu.make_async_copy(wword_hbm.at[idx_v.at[l]], gb[par],
                                     gsem[par])

    def write_desc(l, par):
        return pltpu.make_async_copy(
            ob[par],
            out_hbm.at[l // 2, pl.ds(b0, BPW), pl.ds(DIM * (l % 2), DIM)],
            wsem[par])

    gather_desc(0, 0).start()
    gather_desc(1, 1).start()

    def step(i, carry):
        for par in range(2):
            l = 2 * i + par
            gather_desc(l, par).wait()
            @pl.when(i >= 1)
            def _():
                write_desc(l - 2, par).wait()
            p = [pos_v[l, pl.ds(LANES * g, LANES)] for g in range(GROUPS)]
            gbuf = gb[par]
            obuf = ob[par]

            def add_block(r4, carry2):
                for rr in range(RUNROLL):
                    r = r4 * RUNROLL + rr
                    for g in range(GROUPS):
                        sl = pl.ds(LANES * g, LANES)
                        obuf[r, sl] = gbuf[r, sl] + p[g]
                return carry2

            lax.fori_loop(0, BPW // RUNROLL, add_block, 0)
            @pl.when(l + 2 < SEQ)
            def _():
                gather_desc(l + 2, par).start()
            write_desc(l, par).start()
        return carry

    lax.fori_loop(0, SEQ // 2, step, 0)
    write_desc(SEQ - 2, 0).wait()
    write_desc(SEQ - 1, 1).wait()


def _tc_transpose_body(in_ref, out_ref):
    out_ref[...] = in_ref[0].T


@functools.partial(jax.jit, donate_argnums=())
def kernel(x, W_pos, W_word):
    xT = x.T
    pos = W_pos[:SEQ]
    mesh = plsc.VectorSubcoreMesh(core_axis_name="c", subcore_axis_name="s",
                                  num_cores=NC, num_subcores=NS)
    run = pl.kernel(
        _sc_body,
        out_type=jax.ShapeDtypeStruct((PAIRS, BATCH, 2 * DIM), jnp.float32),
        mesh=mesh,
        scratch_types=[
            pltpu.VMEM((SEQ, BPW), jnp.int32),
            pltpu.VMEM((SEQ, DIM), jnp.float32),
            pltpu.VMEM((BPW, DIM), jnp.float32),
            pltpu.VMEM((BPW, DIM), jnp.float32),
            pltpu.VMEM((BPW, DIM), jnp.float32),
            pltpu.VMEM((BPW, DIM), jnp.float32),
            pltpu.SemaphoreType.DMA,
            pltpu.SemaphoreType.DMA,
            pltpu.SemaphoreType.DMA,
            pltpu.SemaphoreType.DMA,
        ],
        compiler_params=pltpu.CompilerParams(use_tc_tiling_on_sc=False),
    )
    mid = run(xT, pos, W_word)

    outT = pl.pallas_call(
        _tc_transpose_body,
        grid=(PAIRS,),
        in_specs=[pl.BlockSpec((1, BATCH, 2 * DIM), lambda P: (P, 0, 0))],
        out_specs=pl.BlockSpec((2 * DIM, BATCH), lambda P: (P, 0)),
        out_shape=jax.ShapeDtypeStruct((PAIRS * 2 * DIM, BATCH), jnp.float32),
    )(mid)

    return jnp.transpose(outT.reshape(SEQ, DIM, BATCH), (2, 0, 1))


# speedup vs baseline: 1.7728x; 1.0291x over previous
import functools

import jax
import jax.numpy as jnp
from jax import lax
from jax.experimental import pallas as pl
from jax.experimental.pallas import tpu as pltpu
from jax.experimental.pallas import tpu_sc as plsc

NC = 2
NS = 16
NW = NC * NS

BATCH = 4096
SEQ = 200
DIM = 64
PAIRS = SEQ // 2
BPW = BATCH // NW
LANES = 16
GROUPS = DIM // LANES
RUNROLL = 4
BB = 512


def _sc_body(xT_hbm, pos_hbm, wword_hbm, out_hbm,
             idx_v, pos_v, gb0, gb1, ob0, ob1,
             gsem0, gsem1, wsem0, wsem1):
    c = lax.axis_index("c")
    s = lax.axis_index("s")
    wid = s * NC + c
    b0 = wid * BPW

    gb = (gb0, gb1)
    ob = (ob0, ob1)
    gsem = (gsem0, gsem1)
    wsem = (wsem0, wsem1)

    pltpu.sync_copy(xT_hbm.at[:, pl.ds(b0, BPW)], idx_v)
    pltpu.sync_copy(pos_hbm, pos_v)

    def gather_desc(l, par):
        return pltpu.make_async_copy(wword_hbm.at[idx_v.at[l]], gb[par],
                                     gsem[par])

    def write_desc(l, par):
        return pltpu.make_async_copy(
            ob[par],
            out_hbm.at[l // 2, pl.ds(b0, BPW), pl.ds(DIM * (l % 2), DIM)],
            wsem[par])

    gather_desc(0, 0).start()
    gather_desc(1, 1).start()

    def step(i, carry):
        for par in range(2):
            l = 2 * i + par
            gather_desc(l, par).wait()
            @pl.when(i >= 1)
            def _():
                write_desc(l - 2, par).wait()
            p = [pos_v[l, pl.ds(LANES * g, LANES)] for g in range(GROUPS)]
            gbuf = gb[par]
            obuf = ob[par]

            def add_block(r4, carry2):
                for rr in range(RUNROLL):
                    r = r4 * RUNROLL + rr
                    for g in range(GROUPS):
                        sl = pl.ds(LANES * g, LANES)
                        obuf[r, sl] = gbuf[r, sl] + p[g]
                return carry2

            lax.fori_loop(0, BPW // RUNROLL, add_block, 0)
            @pl.when(l + 2 < SEQ)
            def _():
                gather_desc(l + 2, par).start()
            write_desc(l, par).start()
        return carry

    lax.fori_loop(0, SEQ // 2, step, 0)
    write_desc(SEQ - 2, 0).wait()
    write_desc(SEQ - 1, 1).wait()


TPB = 2


def _tc_transpose_body(in_ref, out_ref):
    for k in range(TPB):
        out_ref[pl.ds(2 * DIM * k, 2 * DIM), :] = in_ref[k].T


@functools.partial(jax.jit, donate_argnums=())
def kernel(x, W_pos, W_word):
    xT = x.T
    pos = W_pos[:SEQ]
    mesh = plsc.VectorSubcoreMesh(core_axis_name="c", subcore_axis_name="s",
                                  num_cores=NC, num_subcores=NS)
    run = pl.kernel(
        _sc_body,
        out_type=jax.ShapeDtypeStruct((PAIRS, BATCH, 2 * DIM), jnp.float32),
        mesh=mesh,
        scratch_types=[
            pltpu.VMEM((SEQ, BPW), jnp.int32),
            pltpu.VMEM((SEQ, DIM), jnp.float32),
            pltpu.VMEM((BPW, DIM), jnp.float32),
            pltpu.VMEM((BPW, DIM), jnp.float32),
            pltpu.VMEM((BPW, DIM), jnp.float32),
            pltpu.VMEM((BPW, DIM), jnp.float32),
            pltpu.SemaphoreType.DMA,
            pltpu.SemaphoreType.DMA,
            pltpu.SemaphoreType.DMA,
            pltpu.SemaphoreType.DMA,
        ],
        compiler_params=pltpu.CompilerParams(use_tc_tiling_on_sc=False),
    )
    mid = run(xT, pos, W_word)

    outT = pl.pallas_call(
        _tc_transpose_body,
        grid=(PAIRS // TPB,),
        in_specs=[pl.BlockSpec((TPB, BATCH, 2 * DIM), lambda P: (P, 0, 0))],
        out_specs=pl.BlockSpec((TPB * 2 * DIM, BATCH), lambda P: (P, 0)),
        out_shape=jax.ShapeDtypeStruct((PAIRS * 2 * DIM, BATCH), jnp.float32),
    )(mid)

    return jnp.transpose(outT.reshape(SEQ, DIM, BATCH), (2, 0, 1))


# speedup vs baseline: 1.7758x; 1.0017x over previous
import functools

import jax
import jax.numpy as jnp
from jax import lax
from jax.experimental import pallas as pl
from jax.experimental.pallas import tpu as pltpu
from jax.experimental.pallas import tpu_sc as plsc

NC = 2
NS = 16
NW = NC * NS

BATCH = 4096
SEQ = 200
DIM = 64
PAIRS = SEQ // 2
BPW = BATCH // NW
LANES = 16
GROUPS = DIM // LANES
RUNROLL = 4
BB = 512


def _sc_body(xT_hbm, pos_hbm, wword_hbm, out_hbm,
             idx_v, pos_v, gb0, gb1, ob0, ob1,
             gsem0, gsem1, wsem0, wsem1):
    c = lax.axis_index("c")
    s = lax.axis_index("s")
    wid = s * NC + c
    b0 = wid * BPW

    gb = (gb0, gb1)
    ob = (ob0, ob1)
    gsem = (gsem0, gsem1)
    wsem = (wsem0, wsem1)

    pltpu.sync_copy(xT_hbm.at[:, pl.ds(b0, BPW)], idx_v)
    pltpu.sync_copy(pos_hbm, pos_v)

    def gather_desc(l, par):
        return pltpu.make_async_copy(wword_hbm.at[idx_v.at[l]], gb[par],
                                     gsem[par])

    def write_desc(l, par):
        return pltpu.make_async_copy(
            ob[par],
            out_hbm.at[l // 2, pl.ds(b0, BPW), pl.ds(DIM * (l % 2), DIM)],
            wsem[par])

    gather_desc(0, 0).start()
    gather_desc(1, 1).start()

    def step(i, carry):
        for par in range(2):
            l = 2 * i + par
            gather_desc(l, par).wait()
            @pl.when(i >= 1)
            def _():
                write_desc(l - 2, par).wait()
            p = [pos_v[l, pl.ds(LANES * g, LANES)] for g in range(GROUPS)]
            gbuf = gb[par]
            obuf = ob[par]

            def add_block(r4, carry2):
                for rr in range(RUNROLL):
                    r = r4 * RUNROLL + rr
                    for g in range(GROUPS):
                        sl = pl.ds(LANES * g, LANES)
                        obuf[r, sl] = gbuf[r, sl] + p[g]
                return carry2

            lax.fori_loop(0, BPW // RUNROLL, add_block, 0)
            @pl.when(l + 2 < SEQ)
            def _():
                gather_desc(l + 2, par).start()
            write_desc(l, par).start()
        return carry

    lax.fori_loop(0, SEQ // 2, step, 0)
    write_desc(SEQ - 2, 0).wait()
    write_desc(SEQ - 1, 1).wait()


TPB = 4


def _tc_transpose_body(in_ref, out_ref):
    for k in range(TPB):
        out_ref[pl.ds(2 * DIM * k, 2 * DIM), :] = in_ref[k].T


@functools.partial(jax.jit, donate_argnums=())
def kernel(x, W_pos, W_word):
    xT = x.T
    pos = W_pos[:SEQ]
    mesh = plsc.VectorSubcoreMesh(core_axis_name="c", subcore_axis_name="s",
                                  num_cores=NC, num_subcores=NS)
    run = pl.kernel(
        _sc_body,
        out_type=jax.ShapeDtypeStruct((PAIRS, BATCH, 2 * DIM), jnp.float32),
        mesh=mesh,
        scratch_types=[
            pltpu.VMEM((SEQ, BPW), jnp.int32),
            pltpu.VMEM((SEQ, DIM), jnp.float32),
            pltpu.VMEM((BPW, DIM), jnp.float32),
            pltpu.VMEM((BPW, DIM), jnp.float32),
            pltpu.VMEM((BPW, DIM), jnp.float32),
            pltpu.VMEM((BPW, DIM), jnp.float32),
            pltpu.SemaphoreType.DMA,
            pltpu.SemaphoreType.DMA,
            pltpu.SemaphoreType.DMA,
            pltpu.SemaphoreType.DMA,
        ],
        compiler_params=pltpu.CompilerParams(use_tc_tiling_on_sc=False),
    )
    mid = run(xT, pos, W_word)

    outT = pl.pallas_call(
        _tc_transpose_body,
        grid=(PAIRS // TPB,),
        in_specs=[pl.BlockSpec((TPB, BATCH, 2 * DIM), lambda P: (P, 0, 0))],
        out_specs=pl.BlockSpec((TPB * 2 * DIM, BATCH), lambda P: (P, 0)),
        out_shape=jax.ShapeDtypeStruct((PAIRS * 2 * DIM, BATCH), jnp.float32),
    )(mid)

    return jnp.transpose(outT.reshape(SEQ, DIM, BATCH), (2, 0, 1))


# speedup vs baseline: 1.7779x; 1.0011x over previous
import functools

import jax
import jax.numpy as jnp
from jax import lax
from jax.experimental import pallas as pl
from jax.experimental.pallas import tpu as pltpu
from jax.experimental.pallas import tpu_sc as plsc

NC = 2
NS = 16
NW = NC * NS

BATCH = 4096
SEQ = 200
DIM = 64
PAIRS = SEQ // 2
BPW = BATCH // NW
LANES = 16
GROUPS = DIM // LANES
RUNROLL = 4


def _sc_body(xT_hbm, pos_hbm, wword_hbm, out_hbm,
             idx_v, pos_v, gb0, gb1, ob0, ob1,
             gsem0, gsem1, wsem0, wsem1):
    c = lax.axis_index("c")
    s = lax.axis_index("s")
    wid = s * NC + c
    b0 = wid * BPW

    gb = (gb0, gb1)
    ob = (ob0, ob1)
    gsem = (gsem0, gsem1)
    wsem = (wsem0, wsem1)

    pltpu.sync_copy(xT_hbm.at[:, pl.ds(b0, BPW)], idx_v)
    pltpu.sync_copy(pos_hbm, pos_v)

    def gather_desc(l, par):
        return pltpu.make_async_copy(wword_hbm.at[idx_v.at[l]], gb[par],
                                     gsem[par])

    def write_desc(l, par):
        return pltpu.make_async_copy(
            ob[par],
            out_hbm.at[l // 2, pl.ds(b0, BPW), pl.ds(DIM * (l % 2), DIM)],
            wsem[par])

    gather_desc(0, 0).start()
    gather_desc(1, 1).start()

    def step(i, carry):
        for par in range(2):
            l = 2 * i + par
            gather_desc(l, par).wait()
            @pl.when(i >= 1)
            def _():
                write_desc(l - 2, par).wait()
            p = [pos_v[l, pl.ds(LANES * g, LANES)] for g in range(GROUPS)]
            gbuf = gb[par]
            obuf = ob[par]

            def add_block(r4, carry2):
                for rr in range(RUNROLL):
                    r = r4 * RUNROLL + rr
                    for g in range(GROUPS):
                        sl = pl.ds(LANES * g, LANES)
                        obuf[r, sl] = gbuf[r, sl] + p[g]
                return carry2

            lax.fori_loop(0, BPW // RUNROLL, add_block, 0)
            @pl.when(l + 2 < SEQ)
            def _():
                gather_desc(l + 2, par).start()
            write_desc(l, par).start()
        return carry

    lax.fori_loop(0, SEQ // 2, step, 0)
    write_desc(SEQ - 2, 0).wait()
    write_desc(SEQ - 1, 1).wait()


TPB = 4


def _tc_transpose_body(in_ref, out_ref):
    for k in range(TPB):
        out_ref[pl.ds(2 * DIM * k, 2 * DIM), :] = in_ref[k].T


@functools.partial(jax.jit, donate_argnums=())
def kernel(x, W_pos, W_word):
    xT = x.T
    pos = W_pos[:SEQ]
    mesh = plsc.VectorSubcoreMesh(core_axis_name="c", subcore_axis_name="s",
                                  num_cores=NC, num_subcores=NS)
    run = pl.kernel(
        _sc_body,
        out_type=jax.ShapeDtypeStruct((PAIRS, BATCH, 2 * DIM), jnp.float32),
        mesh=mesh,
        scratch_types=[
            pltpu.VMEM((SEQ, BPW), jnp.int32),
            pltpu.VMEM((SEQ, DIM), jnp.float32),
            pltpu.VMEM((BPW, DIM), jnp.float32),
            pltpu.VMEM((BPW, DIM), jnp.float32),
            pltpu.VMEM((BPW, DIM), jnp.float32),
            pltpu.VMEM((BPW, DIM), jnp.float32),
            pltpu.SemaphoreType.DMA,
            pltpu.SemaphoreType.DMA,
            pltpu.SemaphoreType.DMA,
            pltpu.SemaphoreType.DMA,
        ],
        compiler_params=pltpu.CompilerParams(use_tc_tiling_on_sc=False),
    )
    mid = run(xT, pos, W_word)

    outT = pl.pallas_call(
        _tc_transpose_body,
        grid=(PAIRS // TPB,),
        in_specs=[pl.BlockSpec((TPB, BATCH, 2 * DIM), lambda P: (P, 0, 0))],
        out_specs=pl.BlockSpec((TPB * 2 * DIM, BATCH), lambda P: (P, 0)),
        out_shape=jax.ShapeDtypeStruct((PAIRS * 2 * DIM, BATCH), jnp.float32),
    )(mid)

    return jnp.transpose(outT.reshape(SEQ, DIM, BATCH), (2, 0, 1))
